# baseline (speedup 1.0000x reference)
import functools
from math import sqrt

import jax
import jax.numpy as jnp
from jax import lax
from jax.experimental import pallas as pl
from jax.experimental.pallas import tpu as pltpu
from jax.experimental.pallas import tpu_sc as plsc

N_PAD = 10240
D = 128
E_PAD = 163840
EDGES_PER_W = E_PAD // 32
CHUNK = 32
CHUNKS = EDGES_PER_W // CHUNK
ROWS_PER_SUB = N_PAD // 16
INV_SQRT_D = 1.0 / sqrt(D)


def _mm8_body(x_ref, w_ref, b_ref, *out_refs):
    h = jnp.dot(x_ref[...], w_ref[...], preferred_element_type=jnp.float32)
    h = h + b_ref[0][None, :]
    for k in range(8):
        out_refs[k][...] = h[:, k * D:(k + 1) * D]


def _mm8(x, wt, b8):
    bm = N_PAD // 8
    return pl.pallas_call(
        _mm8_body,
        grid=(8,),
        in_specs=[
            pl.BlockSpec((bm, D), lambda i: (i, 0)),
            pl.BlockSpec((D, 8 * D), lambda i: (0, 0)),
            pl.BlockSpec((8, 8 * D), lambda i: (0, 0)),
        ],
        out_specs=[pl.BlockSpec((bm, D), lambda i: (i, 0)) for _ in range(8)],
        out_shape=[jax.ShapeDtypeStruct((N_PAD, D), jnp.float32) for _ in range(8)],
    )(x, wt, b8)


def _mm1_body(x_ref, w_ref, b_ref, out_ref):
    h = jnp.dot(x_ref[...], w_ref[...], preferred_element_type=jnp.float32)
    out_ref[...] = h + b_ref[0][None, :]


def _mm1(x, wt, b8):
    bm = N_PAD // 8
    return pl.pallas_call(
        _mm1_body,
        grid=(8,),
        in_specs=[
            pl.BlockSpec((bm, D), lambda i: (i, 0)),
            pl.BlockSpec((D, D), lambda i: (0, 0)),
            pl.BlockSpec((8, D), lambda i: (0, 0)),
        ],
        out_specs=pl.BlockSpec((bm, D), lambda i: (i, 0)),
        out_shape=jax.ShapeDtypeStruct((N_PAD, D), jnp.float32),
    )(x, wt, b8)


def _combine_body(oue_ref, due_ref, oap_ref, dap_ref, x1_ref, x5_ref,
                  g_ref, b_ref, out_ref):
    due = due_ref[0] + due_ref[1]
    dap = dap_ref[0] + dap_ref[1]
    due = jnp.where(due == 0.0, 1.0, due)
    dap = jnp.where(dap == 0.0, 1.0, dap)
    oue = oue_ref[0] + oue_ref[1]
    oap = oap_ref[0] + oap_ref[1]
    o = (oue / due[:, None] + x1_ref[...] + oap / dap[:, None] + x5_ref[...])
    o = jnp.maximum(o, 0.0)
    m = jnp.mean(o, axis=1, keepdims=True)
    v = jnp.mean((o - m) * (o - m), axis=1, keepdims=True)
    out_ref[...] = (o - m) / jnp.sqrt(v + 1e-5) * g_ref[0][None, :] + b_ref[0][None, :]


def _combine(oue, due, oap, dap, x1, x5, g8, b8):
    bm = N_PAD // 8
    return pl.pallas_call(
        _combine_body,
        grid=(8,),
        in_specs=[
            pl.BlockSpec((2, bm, D), lambda i: (0, i, 0)),
            pl.BlockSpec((2, bm), lambda i: (0, i)),
            pl.BlockSpec((2, bm, D), lambda i: (0, i, 0)),
            pl.BlockSpec((2, bm), lambda i: (0, i)),
            pl.BlockSpec((bm, D), lambda i: (i, 0)),
            pl.BlockSpec((bm, D), lambda i: (i, 0)),
            pl.BlockSpec((8, D), lambda i: (0, 0)),
            pl.BlockSpec((8, D), lambda i: (0, 0)),
        ],
        out_specs=pl.BlockSpec((bm, D), lambda i: (i, 0)),
        out_shape=jax.ShapeDtypeStruct((N_PAD, D), jnp.float32),
    )(oue, due, oap, dap, x1, x5, g8, b8)


def _edge_body(a_hbm, b_hbm, c_hbm, idx_hbm,
               out_hbm, den_hbm,
               idxs, arows, brows, crows, cscaled, svec, zbuf,
               acc, den_acc,
               sem_g0, sem_g1, sem_s0, sem_s1):
    c_ax = lax.axis_index("c")
    s_ax = lax.axis_index("s")
    wid = s_ax * 2 + c_ax
    rbase = s_ax * ROWS_PER_SUB
    dbase = s_ax * (N_PAD // 16)
    sem_g = (sem_g0, sem_g1)
    sem_s = (sem_s0, sem_s1)

    z16 = jnp.zeros((16,), jnp.float32)
    lane = jnp.arange(16, dtype=jnp.int32)

    pltpu.sync_copy(idx_hbm.at[wid], idxs)

    def _zrow(i, carry):
        for t in range(8):
            cscaled[0][i, pl.ds(16 * t, 16)] = z16
        return carry
    lax.fori_loop(0, CHUNK, _zrow, 0)

    def _zden(i, carry):
        zbuf[pl.ds(i * 16, 16)] = z16
        return carry
    lax.fori_loop(0, (N_PAD // 16) // 16, _zden, 0)

    for q in range(ROWS_PER_SUB // CHUNK):
        pltpu.sync_copy(cscaled[0], acc.at[pl.ds(rbase + q * CHUNK, CHUNK)])
    pltpu.sync_copy(zbuf, den_acc.at[pl.ds(dbase, N_PAD // 16)])

    plsc.subcore_barrier()

    def _issue_gathers(g, b):
        pltpu.async_copy(a_hbm.at[idxs.at[2 * g]], arows[b], sem_g[b])
        pltpu.async_copy(b_hbm.at[idxs.at[2 * g + 1]], brows[b], sem_g[b])
        pltpu.async_copy(c_hbm.at[idxs.at[2 * g + 1]], crows[b], sem_g[b])

    def _wait_gathers(g, b):
        pltpu.make_async_copy(a_hbm.at[idxs.at[2 * g]], arows[b], sem_g[b]).wait()
        pltpu.make_async_copy(b_hbm.at[idxs.at[2 * g + 1]], brows[b], sem_g[b]).wait()
        pltpu.make_async_copy(c_hbm.at[idxs.at[2 * g + 1]], crows[b], sem_g[b]).wait()

    def _issue_scatters(g, b):
        pltpu.async_copy(cscaled[b], acc.at[idxs.at[2 * g]], sem_s[b], add=True)
        pltpu.async_copy(svec[b], den_acc.at[idxs.at[2 * g]], sem_s[b], add=True)

    def _wait_scatters(g, b):
        pltpu.make_async_copy(cscaled[b], acc.at[idxs.at[2 * g]], sem_s[b]).wait()
        pltpu.make_async_copy(svec[b], den_acc.at[idxs.at[2 * g]], sem_s[b]).wait()

    def _compute(b):
        for grp in range(CHUNK // 16):
            def _edge4(i, s16):
                for k in range(4):
                    l = i * 4 + k
                    e = grp * 16 + l
                    pacc = arows[b][e, pl.ds(0, 16)] * brows[b][e, pl.ds(0, 16)]
                    for t in range(1, 8):
                        pacc = pacc + (arows[b][e, pl.ds(16 * t, 16)]
                                       * brows[b][e, pl.ds(16 * t, 16)])
                    dot = jnp.sum(pacc)
                    sv = jnp.exp((jnp.zeros((16,), jnp.float32) + dot)
                                 * INV_SQRT_D)
                    for t in range(8):
                        cscaled[b][e, pl.ds(16 * t, 16)] = (
                            crows[b][e, pl.ds(16 * t, 16)] * sv)
                    s16 = jnp.where(lane == l, sv, s16)
                return s16
            s16 = lax.fori_loop(0, 4, _edge4, jnp.zeros((16,), jnp.float32))
            svec[b][pl.ds(16 * grp, 16)] = s16

    _issue_gathers(0, 0)

    def _pair(gp, carry):
        for b in range(2):
            g = 2 * gp + b
            _wait_gathers(g, b)
            if b == 0:
                _issue_gathers(g + 1, 1)
            else:
                @pl.when(gp < CHUNKS // 2 - 1)
                def _():
                    _issue_gathers(g + 1, 0)

            @pl.when(gp >= 1)
            def _():
                _wait_scatters(g - 2, b)
            _compute(b)
            _issue_scatters(g, b)
        return carry
    lax.fori_loop(0, CHUNKS // 2, _pair, 0)
    _wait_scatters(CHUNKS - 2, 0)
    _wait_scatters(CHUNKS - 1, 1)

    plsc.subcore_barrier()
    pltpu.sync_copy(acc.at[pl.ds(rbase, ROWS_PER_SUB)],
                    out_hbm.at[c_ax, pl.ds(rbase, ROWS_PER_SUB)])
    pltpu.sync_copy(den_acc.at[pl.ds(dbase, N_PAD // 16)],
                    den_hbm.at[c_ax, pl.ds(dbase, N_PAD // 16)])


def _edge_pass(a, b, c, idx_packed):
    mesh = plsc.VectorSubcoreMesh(core_axis_name="c", subcore_axis_name="s")
    fn = functools.partial(
        pl.kernel,
        mesh=mesh,
        compiler_params=pltpu.CompilerParams(
            needs_layout_passes=False, use_tc_tiling_on_sc=False),
        out_type=[
            jax.ShapeDtypeStruct((2, N_PAD, D), jnp.float32),
            jax.ShapeDtypeStruct((2, N_PAD), jnp.float32),
        ],
        scratch_types=[
            pltpu.VMEM((2 * CHUNKS, CHUNK), jnp.int32),
            [pltpu.VMEM((CHUNK, D), jnp.float32) for _ in range(2)],
            [pltpu.VMEM((CHUNK, D), jnp.float32) for _ in range(2)],
            [pltpu.VMEM((CHUNK, D), jnp.float32) for _ in range(2)],
            [pltpu.VMEM((CHUNK, D), jnp.float32) for _ in range(2)],
            [pltpu.VMEM((CHUNK,), jnp.float32) for _ in range(2)],
            pltpu.VMEM((N_PAD // 16,), jnp.float32),
            pltpu.VMEM_SHARED((N_PAD, D), jnp.float32),
            pltpu.VMEM_SHARED((N_PAD,), jnp.float32),
            pltpu.SemaphoreType.DMA,
            pltpu.SemaphoreType.DMA,
            pltpu.SemaphoreType.DMA,
            pltpu.SemaphoreType.DMA,
        ],
    )(_edge_body)
    return fn(a, b, c, idx_packed)


def kernel(x, edge_index_ue, edge_index_ap, params):
    n, d = x.shape
    xp = jnp.zeros((N_PAD, D), jnp.float32).at[:n].set(x)

    pad_n = E_PAD - edge_index_ue.shape[1]
    pad_ri = n + (jnp.arange(pad_n, dtype=jnp.int32) % (N_PAD - n))
    pad_rj = jnp.zeros((pad_n,), jnp.int32)

    def pack_edges(ei):
        rj = jnp.concatenate([ei[0], pad_rj]).reshape(32, CHUNKS, CHUNK)
        ri = jnp.concatenate([ei[1], pad_ri]).reshape(32, CHUNKS, CHUNK)
        return jnp.stack([ri, rj], axis=2).reshape(32, 2 * CHUNKS, CHUNK)

    idx_ue = pack_edges(edge_index_ue)
    idx_ap = pack_edges(edge_index_ap)

    for lp in params['layers']:
        wt = jnp.concatenate([lp['W' + str(k)] for k in range(1, 9)], axis=0).T
        b8 = jnp.broadcast_to(
            jnp.concatenate([lp['b' + str(k)] for k in range(1, 9)]), (8, 8 * D))
        x1, x2, x3, x4, x5, x6, x7, x8 = _mm8(xp, wt, b8)
        oue, due = _edge_pass(x3, x4, x2, idx_ue)
        oap, dap = _edge_pass(x7, x8, x6, idx_ap)
        g8 = jnp.broadcast_to(lp['gamma'], (8, D))
        be8 = jnp.broadcast_to(lp['beta'], (8, D))
        xp = _combine(oue, due, oap, dap, x1, x5, g8, be8)

    wlin_t = jnp.zeros((D, D), jnp.float32).at[:, :6].set(params['Wlin'].T)
    blin8 = jnp.broadcast_to(
        jnp.zeros((D,), jnp.float32).at[:6].set(params['blin']), (8, D))
    out = _mm1(xp, wlin_t, blin8)
    return out[:n, :6]


# speedup vs baseline: 1.0110x; 1.0110x over previous
import functools
from math import sqrt

import jax
import jax.numpy as jnp
from jax import lax
from jax.experimental import pallas as pl
from jax.experimental.pallas import tpu as pltpu
from jax.experimental.pallas import tpu_sc as plsc

N_PAD = 10240
D = 128
E_PAD = 163840
EDGES_PER_W = E_PAD // 32
CHUNK = 32
CHUNKS = EDGES_PER_W // CHUNK
ROWS_PER_SUB = N_PAD // 16
INV_SQRT_D = 1.0 / sqrt(D)


def _mm8_body(x_ref, w_ref, b_ref, *out_refs):
    h = jnp.dot(x_ref[...], w_ref[...], preferred_element_type=jnp.float32)
    h = h + b_ref[0][None, :]
    for k in range(4):
        out_refs[k][...] = h[:, k * D:(k + 1) * D]
    out_refs[4][...] = h[:, 4 * D:6 * D]
    out_refs[5][...] = h[:, 6 * D:8 * D]


def _mm8(x, wt, b8):
    bm = N_PAD // 8
    return pl.pallas_call(
        _mm8_body,
        grid=(8,),
        in_specs=[
            pl.BlockSpec((bm, D), lambda i: (i, 0)),
            pl.BlockSpec((D, 8 * D), lambda i: (0, 0)),
            pl.BlockSpec((8, 8 * D), lambda i: (0, 0)),
        ],
        out_specs=(
            [pl.BlockSpec((bm, D), lambda i: (i, 0)) for _ in range(4)]
            + [pl.BlockSpec((bm, 2 * D), lambda i: (i, 0)) for _ in range(2)]
        ),
        out_shape=(
            [jax.ShapeDtypeStruct((N_PAD, D), jnp.float32) for _ in range(4)]
            + [jax.ShapeDtypeStruct((N_PAD, 2 * D), jnp.float32) for _ in range(2)]
        ),
    )(x, wt, b8)


def _mm1_body(x_ref, w_ref, b_ref, out_ref):
    h = jnp.dot(x_ref[...], w_ref[...], preferred_element_type=jnp.float32)
    out_ref[...] = h + b_ref[0][None, :]


def _mm1(x, wt, b8):
    bm = N_PAD // 8
    return pl.pallas_call(
        _mm1_body,
        grid=(8,),
        in_specs=[
            pl.BlockSpec((bm, D), lambda i: (i, 0)),
            pl.BlockSpec((D, D), lambda i: (0, 0)),
            pl.BlockSpec((8, D), lambda i: (0, 0)),
        ],
        out_specs=pl.BlockSpec((bm, D), lambda i: (i, 0)),
        out_shape=jax.ShapeDtypeStruct((N_PAD, D), jnp.float32),
    )(x, wt, b8)


def _combine_body(oue_ref, due_ref, oap_ref, dap_ref, x1_ref, x5_ref,
                  g_ref, b_ref, out_ref):
    due = due_ref[0] + due_ref[1]
    dap = dap_ref[0] + dap_ref[1]
    due = jnp.where(due == 0.0, 1.0, due)
    dap = jnp.where(dap == 0.0, 1.0, dap)
    oue = oue_ref[0] + oue_ref[1]
    oap = oap_ref[0] + oap_ref[1]
    o = (oue / due[:, None] + x1_ref[...] + oap / dap[:, None] + x5_ref[...])
    o = jnp.maximum(o, 0.0)
    m = jnp.mean(o, axis=1, keepdims=True)
    v = jnp.mean((o - m) * (o - m), axis=1, keepdims=True)
    out_ref[...] = (o - m) / jnp.sqrt(v + 1e-5) * g_ref[0][None, :] + b_ref[0][None, :]


def _combine(oue, due, oap, dap, x1, x5, g8, b8):
    bm = N_PAD // 8
    return pl.pallas_call(
        _combine_body,
        grid=(8,),
        in_specs=[
            pl.BlockSpec((2, bm, D), lambda i: (0, i, 0)),
            pl.BlockSpec((2, bm), lambda i: (0, i)),
            pl.BlockSpec((2, bm, D), lambda i: (0, i, 0)),
            pl.BlockSpec((2, bm), lambda i: (0, i)),
            pl.BlockSpec((bm, D), lambda i: (i, 0)),
            pl.BlockSpec((bm, D), lambda i: (i, 0)),
            pl.BlockSpec((8, D), lambda i: (0, 0)),
            pl.BlockSpec((8, D), lambda i: (0, 0)),
        ],
        out_specs=pl.BlockSpec((bm, D), lambda i: (i, 0)),
        out_shape=jax.ShapeDtypeStruct((N_PAD, D), jnp.float32),
    )(oue, due, oap, dap, x1, x5, g8, b8)


def _edge_body(a_hbm, bc_hbm, idx_hbm,
               out_hbm, den_hbm,
               idxs, arows, bcrows, cscaled, svec, zbuf,
               acc, den_acc,
               sem_g0, sem_g1, sem_s0, sem_s1):
    c_ax = lax.axis_index("c")
    s_ax = lax.axis_index("s")
    wid = s_ax * 2 + c_ax
    rbase = s_ax * ROWS_PER_SUB
    dbase = s_ax * (N_PAD // 16)
    sem_g = (sem_g0, sem_g1)
    sem_s = (sem_s0, sem_s1)

    z16 = jnp.zeros((16,), jnp.float32)
    lane = jnp.arange(16, dtype=jnp.int32)

    pltpu.sync_copy(idx_hbm.at[wid], idxs)

    def _zrow(i, carry):
        for t in range(8):
            cscaled[0][i, pl.ds(16 * t, 16)] = z16
        return carry
    lax.fori_loop(0, CHUNK, _zrow, 0)

    def _zden(i, carry):
        zbuf[pl.ds(i * 16, 16)] = z16
        return carry
    lax.fori_loop(0, (N_PAD // 16) // 16, _zden, 0)

    for q in range(ROWS_PER_SUB // CHUNK):
        pltpu.sync_copy(cscaled[0], acc.at[pl.ds(rbase + q * CHUNK, CHUNK)])
    pltpu.sync_copy(zbuf, den_acc.at[pl.ds(dbase, N_PAD // 16)])

    plsc.subcore_barrier()

    def _issue_gathers(g, b):
        pltpu.async_copy(a_hbm.at[idxs.at[2 * g]], arows[b], sem_g[b])
        pltpu.async_copy(bc_hbm.at[idxs.at[2 * g + 1]], bcrows[b], sem_g[b])

    def _wait_gathers(g, b):
        pltpu.make_async_copy(a_hbm.at[idxs.at[2 * g]], arows[b], sem_g[b]).wait()
        pltpu.make_async_copy(bc_hbm.at[idxs.at[2 * g + 1]], bcrows[b], sem_g[b]).wait()

    def _issue_scatters(g, b):
        pltpu.async_copy(cscaled[b], acc.at[idxs.at[2 * g]], sem_s[b], add=True)
        pltpu.async_copy(svec[b], den_acc.at[idxs.at[2 * g]], sem_s[b], add=True)

    def _wait_scatters(g, b):
        pltpu.make_async_copy(cscaled[b], acc.at[idxs.at[2 * g]], sem_s[b]).wait()
        pltpu.make_async_copy(svec[b], den_acc.at[idxs.at[2 * g]], sem_s[b]).wait()

    def _compute(b):
        for grp in range(CHUNK // 16):
            def _edge4(i, s16):
                for k in range(4):
                    l = i * 4 + k
                    e = grp * 16 + l
                    pacc = arows[b][e, pl.ds(0, 16)] * bcrows[b][e, pl.ds(0, 16)]
                    for t in range(1, 8):
                        pacc = pacc + (arows[b][e, pl.ds(16 * t, 16)]
                                       * bcrows[b][e, pl.ds(16 * t, 16)])
                    dot = jnp.sum(pacc)
                    sv = jnp.exp((jnp.zeros((16,), jnp.float32) + dot)
                                 * INV_SQRT_D)
                    for t in range(8):
                        cscaled[b][e, pl.ds(16 * t, 16)] = (
                            bcrows[b][e, pl.ds(D + 16 * t, 16)] * sv)
                    s16 = jnp.where(lane == l, sv, s16)
                return s16
            s16 = lax.fori_loop(0, 4, _edge4, jnp.zeros((16,), jnp.float32))
            svec[b][pl.ds(16 * grp, 16)] = s16

    _issue_gathers(0, 0)

    def _pair(gp, carry):
        for b in range(2):
            g = 2 * gp + b
            _wait_gathers(g, b)
            if b == 0:
                _issue_gathers(g + 1, 1)
            else:
                @pl.when(gp < CHUNKS // 2 - 1)
                def _():
                    _issue_gathers(g + 1, 0)

            @pl.when(gp >= 1)
            def _():
                _wait_scatters(g - 2, b)
            _compute(b)
            _issue_scatters(g, b)
        return carry
    lax.fori_loop(0, CHUNKS // 2, _pair, 0)
    _wait_scatters(CHUNKS - 2, 0)
    _wait_scatters(CHUNKS - 1, 1)

    plsc.subcore_barrier()
    pltpu.sync_copy(acc.at[pl.ds(rbase, ROWS_PER_SUB)],
                    out_hbm.at[c_ax, pl.ds(rbase, ROWS_PER_SUB)])
    pltpu.sync_copy(den_acc.at[pl.ds(dbase, N_PAD // 16)],
                    den_hbm.at[c_ax, pl.ds(dbase, N_PAD // 16)])


def _edge_pass(a, bc, idx_packed):
    mesh = plsc.VectorSubcoreMesh(core_axis_name="c", subcore_axis_name="s")
    fn = functools.partial(
        pl.kernel,
        mesh=mesh,
        compiler_params=pltpu.CompilerParams(
            needs_layout_passes=False, use_tc_tiling_on_sc=False),
        out_type=[
            jax.ShapeDtypeStruct((2, N_PAD, D), jnp.float32),
            jax.ShapeDtypeStruct((2, N_PAD), jnp.float32),
        ],
        scratch_types=[
            pltpu.VMEM((2 * CHUNKS, CHUNK), jnp.int32),
            [pltpu.VMEM((CHUNK, D), jnp.float32) for _ in range(2)],
            [pltpu.VMEM((CHUNK, 2 * D), jnp.float32) for _ in range(2)],
            [pltpu.VMEM((CHUNK, D), jnp.float32) for _ in range(2)],
            [pltpu.VMEM((CHUNK,), jnp.float32) for _ in range(2)],
            pltpu.VMEM((N_PAD // 16,), jnp.float32),
            pltpu.VMEM_SHARED((N_PAD, D), jnp.float32),
            pltpu.VMEM_SHARED((N_PAD,), jnp.float32),
            pltpu.SemaphoreType.DMA,
            pltpu.SemaphoreType.DMA,
            pltpu.SemaphoreType.DMA,
            pltpu.SemaphoreType.DMA,
        ],
    )(_edge_body)
    return fn(a, bc, idx_packed)


def kernel(x, edge_index_ue, edge_index_ap, params):
    n, d = x.shape
    xp = jnp.zeros((N_PAD, D), jnp.float32).at[:n].set(x)

    pad_n = E_PAD - edge_index_ue.shape[1]
    pad_ri = n + (jnp.arange(pad_n, dtype=jnp.int32) % (N_PAD - n))
    pad_rj = jnp.zeros((pad_n,), jnp.int32)

    def pack_edges(ei):
        rj = jnp.concatenate([ei[0], pad_rj]).reshape(32, CHUNKS, CHUNK)
        ri = jnp.concatenate([ei[1], pad_ri]).reshape(32, CHUNKS, CHUNK)
        return jnp.stack([ri, rj], axis=2).reshape(32, 2 * CHUNKS, CHUNK)

    idx_ue = pack_edges(edge_index_ue)
    idx_ap = pack_edges(edge_index_ap)

    worder = [1, 3, 5, 7, 4, 2, 8, 6]
    for lp in params['layers']:
        wt = jnp.concatenate([lp['W' + str(k)] for k in worder], axis=0).T
        b8 = jnp.broadcast_to(
            jnp.concatenate([lp['b' + str(k)] for k in worder]), (8, 8 * D))
        x1, x3, x5, x7, bc_ue, bc_ap = _mm8(xp, wt, b8)
        oue, due = _edge_pass(x3, bc_ue, idx_ue)
        oap, dap = _edge_pass(x7, bc_ap, idx_ap)
        g8 = jnp.broadcast_to(lp['gamma'], (8, D))
        be8 = jnp.broadcast_to(lp['beta'], (8, D))
        xp = _combine(oue, due, oap, dap, x1, x5, g8, be8)

    wlin_t = jnp.zeros((D, D), jnp.float32).at[:, :6].set(params['Wlin'].T)
    blin8 = jnp.broadcast_to(
        jnp.zeros((D,), jnp.float32).at[:6].set(params['blin']), (8, D))
    out = _mm1(xp, wlin_t, blin8)
    return out[:n, :6]


# speedup vs baseline: 1.0298x; 1.0186x over previous
import functools
from math import sqrt

import jax
import jax.numpy as jnp
from jax import lax
from jax.experimental import pallas as pl
from jax.experimental.pallas import tpu as pltpu
from jax.experimental.pallas import tpu_sc as plsc

N_PAD = 10240
D = 128
E_PAD = 163840
EDGES_PER_W = E_PAD // 32
CHUNK = 32
CHUNKS = EDGES_PER_W // CHUNK
ROWS_PER_SUB = N_PAD // 16
INV_SQRT_D = 1.0 / sqrt(D)


def _mm8_body(x_ref, w_ref, b_ref, *out_refs):
    h = jnp.dot(x_ref[...], w_ref[...], preferred_element_type=jnp.float32)
    h = h + b_ref[0][None, :]
    for k in range(4):
        out_refs[k][...] = h[:, k * D:(k + 1) * D]
    out_refs[4][...] = h[:, 4 * D:6 * D].astype(jnp.bfloat16)
    out_refs[5][...] = h[:, 6 * D:8 * D].astype(jnp.bfloat16)


def _mm8(x, wt, b8):
    bm = N_PAD // 8
    return pl.pallas_call(
        _mm8_body,
        grid=(8,),
        in_specs=[
            pl.BlockSpec((bm, D), lambda i: (i, 0)),
            pl.BlockSpec((D, 8 * D), lambda i: (0, 0)),
            pl.BlockSpec((8, 8 * D), lambda i: (0, 0)),
        ],
        out_specs=(
            [pl.BlockSpec((bm, D), lambda i: (i, 0)) for _ in range(4)]
            + [pl.BlockSpec((bm, 2 * D), lambda i: (i, 0)) for _ in range(2)]
        ),
        out_shape=(
            [jax.ShapeDtypeStruct((N_PAD, D), jnp.float32) for _ in range(4)]
            + [jax.ShapeDtypeStruct((N_PAD, 2 * D), jnp.bfloat16) for _ in range(2)]
        ),
    )(x, wt, b8)


def _mm1_body(x_ref, w_ref, b_ref, out_ref):
    h = jnp.dot(x_ref[...], w_ref[...], preferred_element_type=jnp.float32)
    out_ref[...] = h + b_ref[0][None, :]


def _mm1(x, wt, b8):
    bm = N_PAD // 8
    return pl.pallas_call(
        _mm1_body,
        grid=(8,),
        in_specs=[
            pl.BlockSpec((bm, D), lambda i: (i, 0)),
            pl.BlockSpec((D, D), lambda i: (0, 0)),
            pl.BlockSpec((8, D), lambda i: (0, 0)),
        ],
        out_specs=pl.BlockSpec((bm, D), lambda i: (i, 0)),
        out_shape=jax.ShapeDtypeStruct((N_PAD, D), jnp.float32),
    )(x, wt, b8)


def _combine_body(oue_ref, due_ref, oap_ref, dap_ref, x1_ref, x5_ref,
                  g_ref, b_ref, out_ref):
    due = due_ref[0] + due_ref[1]
    dap = dap_ref[0] + dap_ref[1]
    due = jnp.where(due == 0.0, 1.0, due)
    dap = jnp.where(dap == 0.0, 1.0, dap)
    oue = oue_ref[0] + oue_ref[1]
    oap = oap_ref[0] + oap_ref[1]
    o = (oue / due[:, None] + x1_ref[...] + oap / dap[:, None] + x5_ref[...])
    o = jnp.maximum(o, 0.0)
    m = jnp.mean(o, axis=1, keepdims=True)
    v = jnp.mean((o - m) * (o - m), axis=1, keepdims=True)
    out_ref[...] = (o - m) / jnp.sqrt(v + 1e-5) * g_ref[0][None, :] + b_ref[0][None, :]


def _combine(oue, due, oap, dap, x1, x5, g8, b8):
    bm = N_PAD // 8
    return pl.pallas_call(
        _combine_body,
        grid=(8,),
        in_specs=[
            pl.BlockSpec((2, bm, D), lambda i: (0, i, 0)),
            pl.BlockSpec((2, bm), lambda i: (0, i)),
            pl.BlockSpec((2, bm, D), lambda i: (0, i, 0)),
            pl.BlockSpec((2, bm), lambda i: (0, i)),
            pl.BlockSpec((bm, D), lambda i: (i, 0)),
            pl.BlockSpec((bm, D), lambda i: (i, 0)),
            pl.BlockSpec((8, D), lambda i: (0, 0)),
            pl.BlockSpec((8, D), lambda i: (0, 0)),
        ],
        out_specs=pl.BlockSpec((bm, D), lambda i: (i, 0)),
        out_shape=jax.ShapeDtypeStruct((N_PAD, D), jnp.float32),
    )(oue, due, oap, dap, x1, x5, g8, b8)


def _edge_body(a_hbm, bc_hbm, idx_hbm,
               out_hbm, den_hbm,
               idxs, arows, bcrows, cscaled, svec, zbuf,
               acc, den_acc,
               sem_g0, sem_g1, sem_s0, sem_s1):
    c_ax = lax.axis_index("c")
    s_ax = lax.axis_index("s")
    wid = s_ax * 2 + c_ax
    rbase = s_ax * ROWS_PER_SUB
    dbase = s_ax * (N_PAD // 16)
    sem_g = (sem_g0, sem_g1)
    sem_s = (sem_s0, sem_s1)

    z16 = jnp.zeros((16,), jnp.float32)
    lane = jnp.arange(16, dtype=jnp.int32)

    pltpu.sync_copy(idx_hbm.at[wid], idxs)

    def _zrow(i, carry):
        for t in range(8):
            cscaled[0][i, pl.ds(16 * t, 16)] = z16
        return carry
    lax.fori_loop(0, CHUNK, _zrow, 0)

    def _zden(i, carry):
        zbuf[pl.ds(i * 16, 16)] = z16
        return carry
    lax.fori_loop(0, (N_PAD // 16) // 16, _zden, 0)

    for q in range(ROWS_PER_SUB // CHUNK):
        pltpu.sync_copy(cscaled[0], acc.at[pl.ds(rbase + q * CHUNK, CHUNK)])
    pltpu.sync_copy(zbuf, den_acc.at[pl.ds(dbase, N_PAD // 16)])

    plsc.subcore_barrier()

    def _issue_gathers(g, b):
        pltpu.async_copy(a_hbm.at[idxs.at[2 * g]], arows[b], sem_g[b])
        pltpu.async_copy(bc_hbm.at[idxs.at[2 * g + 1]], bcrows[b], sem_g[b])

    def _wait_gathers(g, b):
        pltpu.make_async_copy(a_hbm.at[idxs.at[2 * g]], arows[b], sem_g[b]).wait()
        pltpu.make_async_copy(bc_hbm.at[idxs.at[2 * g + 1]], bcrows[b], sem_g[b]).wait()

    def _issue_scatters(g, b):
        pltpu.async_copy(cscaled[b], acc.at[idxs.at[2 * g]], sem_s[b], add=True)
        pltpu.async_copy(svec[b], den_acc.at[idxs.at[2 * g]], sem_s[b], add=True)

    def _wait_scatters(g, b):
        pltpu.make_async_copy(cscaled[b], acc.at[idxs.at[2 * g]], sem_s[b]).wait()
        pltpu.make_async_copy(svec[b], den_acc.at[idxs.at[2 * g]], sem_s[b]).wait()

    def _compute(b):
        for grp in range(CHUNK // 16):
            def _edge4(i, s16):
                for k in range(4):
                    l = i * 4 + k
                    e = grp * 16 + l
                    pacc = jnp.zeros((16,), jnp.float32)
                    for t in range(4):
                        w = bcrows[b][e, pl.ds(16 * t, 16)]
                        blo = lax.bitcast_convert_type(w << 16, jnp.float32)
                        bhi = lax.bitcast_convert_type(
                            w & jnp.int32(-65536), jnp.float32)
                        pacc = pacc + arows[b][e, pl.ds(32 * t, 16)] * blo
                        pacc = pacc + arows[b][e, pl.ds(32 * t + 16, 16)] * bhi
                    dot = jnp.sum(pacc)
                    sv = jnp.exp((jnp.zeros((16,), jnp.float32) + dot)
                                 * INV_SQRT_D)
                    for t in range(4):
                        w = bcrows[b][e, pl.ds(64 + 16 * t, 16)]
                        clo = lax.bitcast_convert_type(w << 16, jnp.float32)
                        chi = lax.bitcast_convert_type(
                            w & jnp.int32(-65536), jnp.float32)
                        cscaled[b][e, pl.ds(32 * t, 16)] = clo * sv
                        cscaled[b][e, pl.ds(32 * t + 16, 16)] = chi * sv
                    s16 = jnp.where(lane == l, sv, s16)
                return s16
            s16 = lax.fori_loop(0, 4, _edge4, jnp.zeros((16,), jnp.float32))
            svec[b][pl.ds(16 * grp, 16)] = s16

    _issue_gathers(0, 0)

    def _pair(gp, carry):
        for b in range(2):
            g = 2 * gp + b
            _wait_gathers(g, b)
            if b == 0:
                _issue_gathers(g + 1, 1)
            else:
                @pl.when(gp < CHUNKS // 2 - 1)
                def _():
                    _issue_gathers(g + 1, 0)

            @pl.when(gp >= 1)
            def _():
                _wait_scatters(g - 2, b)
            _compute(b)
            _issue_scatters(g, b)
        return carry
    lax.fori_loop(0, CHUNKS // 2, _pair, 0)
    _wait_scatters(CHUNKS - 2, 0)
    _wait_scatters(CHUNKS - 1, 1)

    plsc.subcore_barrier()
    pltpu.sync_copy(acc.at[pl.ds(rbase, ROWS_PER_SUB)],
                    out_hbm.at[c_ax, pl.ds(rbase, ROWS_PER_SUB)])
    pltpu.sync_copy(den_acc.at[pl.ds(dbase, N_PAD // 16)],
                    den_hbm.at[c_ax, pl.ds(dbase, N_PAD // 16)])


def _edge_pass(a, bc, idx_packed):
    mesh = plsc.VectorSubcoreMesh(core_axis_name="c", subcore_axis_name="s")
    fn = functools.partial(
        pl.kernel,
        mesh=mesh,
        compiler_params=pltpu.CompilerParams(
            needs_layout_passes=False, use_tc_tiling_on_sc=False),
        out_type=[
            jax.ShapeDtypeStruct((2, N_PAD, D), jnp.float32),
            jax.ShapeDtypeStruct((2, N_PAD), jnp.float32),
        ],
        scratch_types=[
            pltpu.VMEM((2 * CHUNKS, CHUNK), jnp.int32),
            [pltpu.VMEM((CHUNK, D), jnp.float32) for _ in range(2)],
            [pltpu.VMEM((CHUNK, D), jnp.int32) for _ in range(2)],
            [pltpu.VMEM((CHUNK, D), jnp.float32) for _ in range(2)],
            [pltpu.VMEM((CHUNK,), jnp.float32) for _ in range(2)],
            pltpu.VMEM((N_PAD // 16,), jnp.float32),
            pltpu.VMEM_SHARED((N_PAD, D), jnp.float32),
            pltpu.VMEM_SHARED((N_PAD,), jnp.float32),
            pltpu.SemaphoreType.DMA,
            pltpu.SemaphoreType.DMA,
            pltpu.SemaphoreType.DMA,
            pltpu.SemaphoreType.DMA,
        ],
    )(_edge_body)
    return fn(a, bc, idx_packed)


def kernel(x, edge_index_ue, edge_index_ap, params):
    n, d = x.shape
    xp = jnp.zeros((N_PAD, D), jnp.float32).at[:n].set(x)

    pad_n = E_PAD - edge_index_ue.shape[1]
    pad_ri = n + (jnp.arange(pad_n, dtype=jnp.int32) % (N_PAD - n))
    pad_rj = jnp.zeros((pad_n,), jnp.int32)

    def pack_edges(ei):
        rj = jnp.concatenate([ei[0], pad_rj]).reshape(32, CHUNKS, CHUNK)
        ri = jnp.concatenate([ei[1], pad_ri]).reshape(32, CHUNKS, CHUNK)
        return jnp.stack([ri, rj], axis=2).reshape(32, 2 * CHUNKS, CHUNK)

    idx_ue = pack_edges(edge_index_ue)
    idx_ap = pack_edges(edge_index_ap)

    perm = jnp.array([32 * t + 16 * p + i
                      for t in range(4) for i in range(16) for p in range(2)],
                     dtype=jnp.int32)
    for lp in params['layers']:
        wperm = [lp['W1'], lp['W3'], lp['W5'], lp['W7'],
                 lp['W4'][perm], lp['W2'][perm], lp['W8'][perm], lp['W6'][perm]]
        bperm = [lp['b1'], lp['b3'], lp['b5'], lp['b7'],
                 lp['b4'][perm], lp['b2'][perm], lp['b8'][perm], lp['b6'][perm]]
        wt = jnp.concatenate(wperm, axis=0).T
        b8 = jnp.broadcast_to(jnp.concatenate(bperm), (8, 8 * D))
        x1, x3, x5, x7, bc_ue, bc_ap = _mm8(xp, wt, b8)
        bc_ue = lax.bitcast_convert_type(
            bc_ue.reshape(N_PAD, D, 2), jnp.int32)
        bc_ap = lax.bitcast_convert_type(
            bc_ap.reshape(N_PAD, D, 2), jnp.int32)
        oue, due = _edge_pass(x3, bc_ue, idx_ue)
        oap, dap = _edge_pass(x7, bc_ap, idx_ap)
        g8 = jnp.broadcast_to(lp['gamma'], (8, D))
        be8 = jnp.broadcast_to(lp['beta'], (8, D))
        xp = _combine(oue, due, oap, dap, x1, x5, g8, be8)

    wlin_t = jnp.zeros((D, D), jnp.float32).at[:, :6].set(params['Wlin'].T)
    blin8 = jnp.broadcast_to(
        jnp.zeros((D,), jnp.float32).at[:6].set(params['blin']), (8, D))
    out = _mm1(xp, wlin_t, blin8)
    return out[:n, :6]
